# baseline (speedup 1.0000x reference)
import numpy as np
import jax
import jax.numpy as jnp
from jax.experimental import pallas as pl
from jax.experimental.pallas import tpu as pltpu

N_CELLS = 64
F_PER_CELL = 16
D = 64
BAND = 8
BB = 16

GROUPS = tuple((o, s) for s in (1, -1) for o in range(1, BAND + 1))


def _dot(a, b):
    return jnp.dot(a, b, preferred_element_type=jnp.float32)


def _shift(x, o, sgn):
    z = jnp.zeros((o, x.shape[1]), x.dtype)
    if sgn > 0:
        return jnp.concatenate([x[o:], z], axis=0)
    return jnp.concatenate([z, x[: x.shape[0] - o]], axis=0)


def _ln(x, g, b):
    m = jnp.mean(x, axis=1, keepdims=True)
    xc = x - m
    v = jnp.mean(xc * xc, axis=1, keepdims=True)
    return xc * jax.lax.rsqrt(v + 1e-5) * g + b


def _fwd_kernel(
    x_ref,
    emb_W, emb_b,
    e0_W1a, e0_W1b, e0_b1, e0_W2, e0_b2,
    n0_W1a, n0_W1b, n0_b1, n0_W2, n0_b2,
    nn_g0, nn_b0, en_g0, en_b0,
    e1_W1a, e1_W1b, e1_W1c, e1_b1, e1_W2, e1_b2,
    n1_W1a, n1_W1b, n1_b1, n1_W2, n1_b2,
    nn_g1, nn_b1,
    h_Wa, h_Wb, h_b, o_W, o_b,
    out_ref,
    e_scr,
):
    R = BB * N_CELLS
    jcell = jax.lax.broadcasted_iota(jnp.int32, (R, 1), 0) % N_CELLS

    def band_mask(o, sgn):
        return (jcell < N_CELLS - o) if sgn > 0 else (jcell >= o)

    node = jnp.maximum(_dot(x_ref[...], emb_W[...]) + emb_b[...], 0.0)

    P = _dot(node, e0_W1a[...])
    Q = _dot(node, e0_W1b[...]) + e0_b1[...]
    W2, b2 = e0_W2[...], e0_b2[...]
    eg, eb = en_g0[...], en_b0[...]
    agg = jnp.zeros((R, D), jnp.float32)
    for gi, (o, sgn) in enumerate(GROUPS):
        h = jnp.maximum(_shift(P, o, sgn) + Q, 0.0)
        e = _dot(h, W2) + b2
        agg = agg + jnp.where(band_mask(o, sgn), e, 0.0)
        e_scr[gi] = _ln(e, eg, eb)
    h = jnp.maximum(_dot(node, n0_W1a[...]) + _dot(agg, n0_W1b[...]) + n0_b1[...], 0.0)
    node = _ln(_dot(h, n0_W2[...]) + n0_b2[...], nn_g0[...], nn_b0[...])

    P = _dot(node, e1_W1a[...])
    Q = _dot(node, e1_W1b[...]) + e1_b1[...]
    W1c, W2, b2 = e1_W1c[...], e1_W2[...], e1_b2[...]
    agg = jnp.zeros((R, D), jnp.float32)
    for gi, (o, sgn) in enumerate(GROUPS):
        h = jnp.maximum(_shift(P, o, sgn) + Q + _dot(e_scr[gi], W1c), 0.0)
        e = _dot(h, W2) + b2
        agg = agg + jnp.where(band_mask(o, sgn), e, 0.0)
    h = jnp.maximum(_dot(node, n1_W1a[...]) + _dot(agg, n1_W1b[...]) + n1_b1[...], 0.0)
    node = _ln(_dot(h, n1_W2[...]) + n1_b2[...], nn_g1[...], nn_b1[...])

    Ph = _dot(node, h_Wa[...])
    Qh = _dot(node, h_Wb[...]) + h_b[...]
    oW, ob = o_W[...], o_b[...]
    cols = []
    for o in range(1, BAND + 1):
        hid = jnp.maximum(_shift(Ph, o, 1) + Qh, 0.0)
        cols.append(jnp.tanh(_dot(hid, oW) + ob))
    out_ref[...] = jnp.concatenate(cols, axis=1)


def _pair_select():
    sel = []
    for i in range(N_CELLS):
        for j in range(max(0, i - BAND), i):
            sel.append(j * BAND + (i - j - 1))
    return np.asarray(sel, dtype=np.int32)


_SEL = _pair_select()


def kernel(observations, edge_index, params):
    B = observations.shape[0]
    x = observations.reshape(B * N_CELLS, F_PER_CELL)
    p = params
    g0, g1 = p['gnn'][0], p['gnn'][1]
    r2 = lambda v: v.reshape(1, -1)
    e0W1, e1W1 = g0['eW'][0], g1['eW'][0]
    n0W1, n1W1 = g0['nW'][0], g1['nW'][0]
    hW = p['hid_W']
    weights = [
        p['emb_W'], r2(p['emb_b']),
        e0W1[:D], e0W1[D:], r2(g0['eb'][0]), g0['eW'][1], r2(g0['eb'][1]),
        n0W1[:D], n0W1[D:], r2(g0['nb'][0]), g0['nW'][1], r2(g0['nb'][1]),
        r2(p['nn_g'][0]), r2(p['nn_b'][0]), r2(p['en_g'][0]), r2(p['en_b'][0]),
        e1W1[:D], e1W1[D:2 * D], e1W1[2 * D:], r2(g1['eb'][0]), g1['eW'][1], r2(g1['eb'][1]),
        n1W1[:D], n1W1[D:], r2(g1['nb'][0]), g1['nW'][1], r2(g1['nb'][1]),
        r2(p['nn_g'][1]), r2(p['nn_b'][1]),
        hW[:D], hW[D:], r2(p['hid_b']), p['out_W'], r2(p['out_b']),
    ]
    R = BB * N_CELLS
    out = pl.pallas_call(
        _fwd_kernel,
        grid=(B // BB,),
        in_specs=[pl.BlockSpec((R, F_PER_CELL), lambda i: (i, 0))]
        + [pl.BlockSpec(w.shape, lambda i, nd=w.ndim: (0,) * nd) for w in weights],
        out_specs=pl.BlockSpec((R, BAND), lambda i: (i, 0)),
        out_shape=jax.ShapeDtypeStruct((B * N_CELLS, BAND), jnp.float32),
        scratch_shapes=[pltpu.VMEM((len(GROUPS), R, D), jnp.float32)],
    )(x, *weights)
    return out.reshape(B, N_CELLS * BAND)[:, _SEL]
